# baseline (speedup 1.0000x reference)
import functools

import jax
import jax.numpy as jnp
from jax import lax
from jax.experimental import pallas as pl
from jax.experimental.pallas import tpu as pltpu
from jax.experimental.pallas import tpu_sc as plsc

N_NODES = 10000
N_EDGES = 320000
D = 128

NC = 2
NS = 16
NW = NC * NS
K = 128
CH_STG = 40
NCH = 80
TOT_CH = NW * NCH
NCHD = NCH
E_PAD = TOT_CH * K
ROWS_PER_TILE = 632
N_ACC = NS * ROWS_PER_TILE

M_BLK = 1000
GRID_M = N_NODES // M_BLK

_sc_mesh = plsc.VectorSubcoreMesh(core_axis_name="c", subcore_axis_name="s")


@functools.partial(
    pl.kernel,
    out_type=jax.ShapeDtypeStruct((NC, N_ACC, 16), jnp.float32),
    mesh=_sc_mesh,
    compiler_params=pltpu.CompilerParams(use_tc_tiling_on_sc=False),
    scratch_types=[
        pltpu.VMEM((NCHD, K), jnp.int32),
        pltpu.VMEM((K, 16), jnp.float32),
        pltpu.VMEM_SHARED((N_ACC, 16), jnp.float32),
    ],
)
def _deg_kernel(colp_hbm, ones_hbm, zeros_hbm, deg_out, col_vm, ones_vm,
                deg_sp):
    c = lax.axis_index("c")
    s = lax.axis_index("s")
    wid = c * NS + s
    pltpu.sync_copy(colp_hbm.at[pl.ds(wid * NCHD, NCHD)], col_vm)
    pltpu.sync_copy(ones_hbm, ones_vm)
    pltpu.sync_copy(zeros_hbm, deg_sp.at[pl.ds(s * ROWS_PER_TILE, ROWS_PER_TILE)])
    plsc.subcore_barrier()

    def body(j, carry):
        pltpu.sync_copy(ones_vm, deg_sp.at[col_vm.at[j]], add=True)
        return carry

    lax.fori_loop(0, NCHD, body, 0)
    plsc.subcore_barrier()
    pltpu.sync_copy(
        deg_sp.at[pl.ds(s * ROWS_PER_TILE, ROWS_PER_TILE)],
        deg_out.at[c, pl.ds(s * ROWS_PER_TILE, ROWS_PER_TILE)],
    )


@functools.partial(
    pl.kernel,
    out_type=jax.ShapeDtypeStruct((NC, N_ACC, D), jnp.float32),
    mesh=_sc_mesh,
    compiler_params=pltpu.CompilerParams(use_tc_tiling_on_sc=False),
    scratch_types=[
        pltpu.VMEM((CH_STG, K), jnp.int32),
        pltpu.VMEM((CH_STG, K), jnp.int32),
        pltpu.VMEM((K, D), jnp.float32),
        pltpu.VMEM((K, D), jnp.float32),
        pltpu.VMEM_SHARED((N_ACC, D), jnp.float32),
        pltpu.SemaphoreType.DMA,
        pltpu.SemaphoreType.DMA,
    ],
)
def _agg_kernel(y_hbm, rowp_hbm, colp_hbm, zeros_hbm, s_out, row_vm, col_vm,
                buf0, buf1, s_sp, sem0, sem1):
    c = lax.axis_index("c")
    s = lax.axis_index("s")
    wid = c * NS + s
    pltpu.sync_copy(zeros_hbm, s_sp.at[pl.ds(s * ROWS_PER_TILE, ROWS_PER_TILE)])
    plsc.subcore_barrier()

    def run_stage(chunk_off):
        pltpu.sync_copy(rowp_hbm.at[pl.ds(chunk_off, CH_STG)], row_vm)
        pltpu.sync_copy(colp_hbm.at[pl.ds(chunk_off, CH_STG)], col_vm)

        pltpu.async_copy(y_hbm.at[row_vm.at[0]], buf0, sem0)

        def body(t, carry):
            j0 = 2 * t
            j1 = 2 * t + 1
            pltpu.async_copy(y_hbm.at[row_vm.at[j1]], buf1, sem1)
            pltpu.make_async_copy(y_hbm.at[row_vm.at[j0]], buf0, sem0).wait()
            pltpu.sync_copy(buf0, s_sp.at[col_vm.at[j0]], add=True)

            @pl.when(j1 + 1 < CH_STG)
            def _():
                pltpu.async_copy(y_hbm.at[row_vm.at[j1 + 1]], buf0, sem0)

            pltpu.make_async_copy(y_hbm.at[row_vm.at[j1]], buf1, sem1).wait()
            pltpu.sync_copy(buf1, s_sp.at[col_vm.at[j1]], add=True)
            return carry

        lax.fori_loop(0, CH_STG // 2, body, 0)

    for h in range(NCH // CH_STG):
        run_stage(wid * NCH + h * CH_STG)

    plsc.subcore_barrier()
    pltpu.sync_copy(
        s_sp.at[pl.ds(s * ROWS_PER_TILE, ROWS_PER_TILE)],
        s_out.at[c, pl.ds(s * ROWS_PER_TILE, ROWS_PER_TILE)],
    )


def _mm_body(cs_ref, w_ref, x_ref):
    x_ref[...] = jnp.dot(cs_ref[...], w_ref[...],
                         preferred_element_type=jnp.float32)


def _mm(cur_state, w):
    return pl.pallas_call(
        _mm_body,
        grid=(GRID_M,),
        in_specs=[
            pl.BlockSpec((M_BLK, D), lambda i: (i, 0)),
            pl.BlockSpec((D, D), lambda i: (0, 0)),
        ],
        out_specs=pl.BlockSpec((M_BLK, D), lambda i: (i, 0)),
        out_shape=jax.ShapeDtypeStruct((N_NODES, D), jnp.float32),
    )(cur_state, w)


def _scale_body(x_ref, degp_ref, y_ref):
    deg = degp_ref[0, :, 0:1] + degp_ref[1, :, 0:1] + 2.0
    dinv = lax.rsqrt(deg)
    y_ref[...] = x_ref[...] * dinv


def _scale(x, degp):
    return pl.pallas_call(
        _scale_body,
        grid=(GRID_M,),
        in_specs=[
            pl.BlockSpec((M_BLK, D), lambda i: (i, 0)),
            pl.BlockSpec((NC, M_BLK, 16), lambda i: (0, i, 0)),
        ],
        out_specs=pl.BlockSpec((M_BLK, D), lambda i: (i, 0)),
        out_shape=jax.ShapeDtypeStruct((N_NODES, D), jnp.float32),
    )(x, degp)


def _combine_body(sp_ref, y_ref, degp_ref, b_ref, out_ref):
    deg = degp_ref[0, :, 0:1] + degp_ref[1, :, 0:1] + 2.0
    dinv = lax.rsqrt(deg)
    acc = sp_ref[0] + sp_ref[1] + 2.0 * y_ref[...]
    out_ref[...] = acc * dinv + b_ref[...]


def _combine(s_acc, y, degp, b2d):
    return pl.pallas_call(
        _combine_body,
        grid=(GRID_M,),
        in_specs=[
            pl.BlockSpec((NC, M_BLK, D), lambda i: (0, i, 0)),
            pl.BlockSpec((M_BLK, D), lambda i: (i, 0)),
            pl.BlockSpec((NC, M_BLK, 16), lambda i: (0, i, 0)),
            pl.BlockSpec((1, D), lambda i: (0, 0)),
        ],
        out_specs=pl.BlockSpec((M_BLK, D), lambda i: (i, 0)),
        out_shape=jax.ShapeDtypeStruct((N_NODES, D), jnp.float32),
    )(s_acc, y, degp, b2d)


def kernel(cur_state, edge_index, W, b):
    row = edge_index[0].astype(jnp.int32)
    col = edge_index[1].astype(jnp.int32)
    pad = E_PAD - N_EDGES
    pad_iota = jnp.arange(pad, dtype=jnp.int32)
    rowp = jnp.concatenate([row, pad_iota % N_NODES])
    colp = jnp.concatenate([col, N_NODES + pad_iota % (N_ACC - N_NODES)])
    rowp = rowp.reshape(TOT_CH, K)
    colp = colp.reshape(TOT_CH, K)

    ones16 = jnp.ones((K, 16), jnp.float32)
    zeros16 = jnp.zeros((ROWS_PER_TILE, 16), jnp.float32)
    zerosd = jnp.zeros((ROWS_PER_TILE, D), jnp.float32)

    degp = _deg_kernel(colp, ones16, zeros16)
    x = _mm(cur_state, W)
    y = _scale(x, degp)
    s_acc = _agg_kernel(y, rowp, colp, zerosd)
    out = _combine(s_acc, y, degp, b.reshape(1, D))
    return out


# speedup vs baseline: 1.0009x; 1.0009x over previous
import functools

import jax
import jax.numpy as jnp
from jax import lax
from jax.experimental import pallas as pl
from jax.experimental.pallas import tpu as pltpu
from jax.experimental.pallas import tpu_sc as plsc

N_NODES = 10000
N_EDGES = 320000
D = 128

NC = 2
NS = 16
NW = NC * NS
K = 128
CH_STG = 40
NCH = 80
TOT_CH = NW * NCH
NCHD = NCH
E_PAD = TOT_CH * K
ROWS_PER_TILE = 632
N_ACC = NS * ROWS_PER_TILE

M_BLK = 1000
GRID_M = N_NODES // M_BLK

_sc_mesh = plsc.VectorSubcoreMesh(core_axis_name="c", subcore_axis_name="s")


@functools.partial(
    pl.kernel,
    out_type=jax.ShapeDtypeStruct((NC, N_ACC, 16), jnp.float32),
    mesh=_sc_mesh,
    compiler_params=pltpu.CompilerParams(use_tc_tiling_on_sc=False),
    scratch_types=[
        pltpu.VMEM((NCHD, K), jnp.int32),
        pltpu.VMEM((K, 16), jnp.float32),
        pltpu.VMEM_SHARED((N_ACC, 16), jnp.float32),
    ],
)
def _deg_kernel(colp_hbm, ones_hbm, zeros_hbm, deg_out, col_vm, ones_vm,
                deg_sp):
    c = lax.axis_index("c")
    s = lax.axis_index("s")
    wid = c * NS + s
    pltpu.sync_copy(colp_hbm.at[pl.ds(wid * NCHD, NCHD)], col_vm)
    pltpu.sync_copy(ones_hbm, ones_vm)
    pltpu.sync_copy(zeros_hbm, deg_sp.at[pl.ds(s * ROWS_PER_TILE, ROWS_PER_TILE)])
    plsc.subcore_barrier()

    def body(j, carry):
        pltpu.sync_copy(ones_vm, deg_sp.at[col_vm.at[j]], add=True)
        return carry

    lax.fori_loop(0, NCHD, body, 0)
    plsc.subcore_barrier()
    pltpu.sync_copy(
        deg_sp.at[pl.ds(s * ROWS_PER_TILE, ROWS_PER_TILE)],
        deg_out.at[c, pl.ds(s * ROWS_PER_TILE, ROWS_PER_TILE)],
    )


@functools.partial(
    pl.kernel,
    out_type=jax.ShapeDtypeStruct((NC, N_ACC, D), jnp.float32),
    mesh=_sc_mesh,
    scratch_types=[
        pltpu.VMEM((CH_STG, K), jnp.int32),
        pltpu.VMEM((CH_STG, K), jnp.int32),
        pltpu.VMEM((K, D), jnp.float32),
        pltpu.VMEM((K, D), jnp.float32),
        pltpu.VMEM_SHARED((N_ACC, D), jnp.float32),
        pltpu.SemaphoreType.DMA,
        pltpu.SemaphoreType.DMA,
    ],
)
def _agg_kernel(y_hbm, rowp_hbm, colp_hbm, zeros_hbm, s_out, row_vm, col_vm,
                buf0, buf1, s_sp, sem0, sem1):
    c = lax.axis_index("c")
    s = lax.axis_index("s")
    wid = c * NS + s
    pltpu.sync_copy(zeros_hbm, s_sp.at[pl.ds(s * ROWS_PER_TILE, ROWS_PER_TILE)])
    plsc.subcore_barrier()

    def run_stage(chunk_off):
        pltpu.sync_copy(rowp_hbm.at[pl.ds(chunk_off, CH_STG)], row_vm)
        pltpu.sync_copy(colp_hbm.at[pl.ds(chunk_off, CH_STG)], col_vm)

        pltpu.async_copy(y_hbm.at[row_vm.at[0]], buf0, sem0)

        def body(t, carry):
            j0 = 2 * t
            j1 = 2 * t + 1
            pltpu.async_copy(y_hbm.at[row_vm.at[j1]], buf1, sem1)
            pltpu.make_async_copy(y_hbm.at[row_vm.at[j0]], buf0, sem0).wait()
            pltpu.sync_copy(buf0, s_sp.at[col_vm.at[j0]], add=True)

            @pl.when(j1 + 1 < CH_STG)
            def _():
                pltpu.async_copy(y_hbm.at[row_vm.at[j1 + 1]], buf0, sem0)

            pltpu.make_async_copy(y_hbm.at[row_vm.at[j1]], buf1, sem1).wait()
            pltpu.sync_copy(buf1, s_sp.at[col_vm.at[j1]], add=True)
            return carry

        lax.fori_loop(0, CH_STG // 2, body, 0)

    for h in range(NCH // CH_STG):
        run_stage(wid * NCH + h * CH_STG)

    plsc.subcore_barrier()
    pltpu.sync_copy(
        s_sp.at[pl.ds(s * ROWS_PER_TILE, ROWS_PER_TILE)],
        s_out.at[c, pl.ds(s * ROWS_PER_TILE, ROWS_PER_TILE)],
    )


def _mm_body(cs_ref, w_ref, x_ref):
    x_ref[...] = jnp.dot(cs_ref[...], w_ref[...],
                         preferred_element_type=jnp.float32)


def _mm(cur_state, w):
    return pl.pallas_call(
        _mm_body,
        grid=(GRID_M,),
        in_specs=[
            pl.BlockSpec((M_BLK, D), lambda i: (i, 0)),
            pl.BlockSpec((D, D), lambda i: (0, 0)),
        ],
        out_specs=pl.BlockSpec((M_BLK, D), lambda i: (i, 0)),
        out_shape=jax.ShapeDtypeStruct((N_NODES, D), jnp.float32),
    )(cur_state, w)


def _scale_body(x_ref, degp_ref, y_ref):
    deg = degp_ref[0, :, 0:1] + degp_ref[1, :, 0:1] + 2.0
    dinv = lax.rsqrt(deg)
    y_ref[...] = x_ref[...] * dinv


def _scale(x, degp):
    return pl.pallas_call(
        _scale_body,
        grid=(GRID_M,),
        in_specs=[
            pl.BlockSpec((M_BLK, D), lambda i: (i, 0)),
            pl.BlockSpec((NC, M_BLK, 16), lambda i: (0, i, 0)),
        ],
        out_specs=pl.BlockSpec((M_BLK, D), lambda i: (i, 0)),
        out_shape=jax.ShapeDtypeStruct((N_NODES, D), jnp.float32),
    )(x, degp)


def _combine_body(sp_ref, y_ref, degp_ref, b_ref, out_ref):
    deg = degp_ref[0, :, 0:1] + degp_ref[1, :, 0:1] + 2.0
    dinv = lax.rsqrt(deg)
    acc = sp_ref[0] + sp_ref[1] + 2.0 * y_ref[...]
    out_ref[...] = acc * dinv + b_ref[...]


def _combine(s_acc, y, degp, b2d):
    return pl.pallas_call(
        _combine_body,
        grid=(GRID_M,),
        in_specs=[
            pl.BlockSpec((NC, M_BLK, D), lambda i: (0, i, 0)),
            pl.BlockSpec((M_BLK, D), lambda i: (i, 0)),
            pl.BlockSpec((NC, M_BLK, 16), lambda i: (0, i, 0)),
            pl.BlockSpec((1, D), lambda i: (0, 0)),
        ],
        out_specs=pl.BlockSpec((M_BLK, D), lambda i: (i, 0)),
        out_shape=jax.ShapeDtypeStruct((N_NODES, D), jnp.float32),
    )(s_acc, y, degp, b2d)


def kernel(cur_state, edge_index, W, b):
    row = edge_index[0].astype(jnp.int32)
    col = edge_index[1].astype(jnp.int32)
    pad = E_PAD - N_EDGES
    pad_iota = jnp.arange(pad, dtype=jnp.int32)
    rowp = jnp.concatenate([row, pad_iota % N_NODES])
    colp = jnp.concatenate([col, N_NODES + pad_iota % (N_ACC - N_NODES)])
    rowp = rowp.reshape(TOT_CH, K)
    colp = colp.reshape(TOT_CH, K)

    ones16 = jnp.ones((K, 16), jnp.float32)
    zeros16 = jnp.zeros((ROWS_PER_TILE, 16), jnp.float32)
    zerosd = jnp.zeros((ROWS_PER_TILE, D), jnp.float32)

    degp = _deg_kernel(colp, ones16, zeros16)
    x = _mm(cur_state, W)
    y = _scale(x, degp)
    s_acc = _agg_kernel(y, rowp, colp, zerosd)
    out = _combine(s_acc, y, degp, b.reshape(1, D))
    return out
